# Initial kernel scaffold; baseline (speedup 1.0000x reference)
import jax
import jax.numpy as jnp
from jax.experimental import pallas as pl


def kernel(x):
    raise NotImplementedError("write your pallas kernel here")


# speedup vs baseline: 83.2748x; 83.2748x over previous
import functools

import jax
import jax.numpy as jnp
from jax import lax
from jax.experimental import pallas as pl
from jax.experimental.pallas import tpu as pltpu
from jax.experimental.pallas import tpu_sc as plsc

B, H, W = 32, 512, 512
P = H * W
NUM_CLASSES = 4
C = 8192
NCHUNK = P // C
LANES = 16

_mesh = plsc.VectorSubcoreMesh(core_axis_name="c", subcore_axis_name="s")


@functools.partial(
    pl.kernel,
    out_type=jax.ShapeDtypeStruct((B, NUM_CLASSES, P), jnp.float32),
    mesh=_mesh,
    scratch_types=[
        pltpu.VMEM((C,), jnp.float32),
        pltpu.VMEM((NUM_CLASSES, C), jnp.float32),
    ],
)
def _onehot_sc(x_hbm, out_hbm, x_v, o_v):
    num_cores = 2
    b = lax.axis_index("s") * num_cores + lax.axis_index("c")

    def chunk_body(j, carry):
        off = j * C
        pltpu.sync_copy(x_hbm.at[b, pl.ds(off, C)], x_v)

        def vec_body(i, carry2):
            sl = pl.ds(i * LANES, LANES)
            v = x_v[sl]
            one = jnp.ones((LANES,), jnp.float32)
            zero = jnp.zeros((LANES,), jnp.float32)
            s0 = jnp.where(v < 0.1, one, zero)
            s1 = jnp.where(v < 1.0, one, zero)
            s2 = jnp.where(v < 2.5, one, zero)
            o_v[0, sl] = s0
            o_v[1, sl] = s1 - s0
            o_v[2, sl] = s2 - s1
            o_v[3, sl] = one - s2
            return carry2

        lax.fori_loop(0, C // LANES, vec_body, 0, unroll=4)
        for cls in range(NUM_CLASSES):
            pltpu.sync_copy(o_v.at[cls], out_hbm.at[b, cls, pl.ds(off, C)])
        return carry

    lax.fori_loop(0, NCHUNK, chunk_body, 0)


def kernel(x):
    x2d = x.reshape(B, P)
    out = _onehot_sc(x2d)
    return out.reshape(B, 1, NUM_CLASSES, H, W)


# speedup vs baseline: 90.9831x; 1.0926x over previous
import functools

import jax
import jax.numpy as jnp
from jax import lax
from jax.experimental import pallas as pl
from jax.experimental.pallas import tpu as pltpu
from jax.experimental.pallas import tpu_sc as plsc

B, H, W = 32, 512, 512
P = H * W
NUM_CLASSES = 4
C = 8192
NCHUNK = P // C
LANES = 16

_mesh = plsc.VectorSubcoreMesh(core_axis_name="c", subcore_axis_name="s")


@functools.partial(
    pl.kernel,
    out_type=jax.ShapeDtypeStruct((B, NUM_CLASSES, P), jnp.float32),
    mesh=_mesh,
    scratch_types=[
        pltpu.VMEM((C,), jnp.float32),
        pltpu.VMEM((C,), jnp.float32),
        pltpu.VMEM((NUM_CLASSES, C), jnp.float32),
        pltpu.VMEM((NUM_CLASSES, C), jnp.float32),
        pltpu.SemaphoreType.DMA,
        pltpu.SemaphoreType.DMA,
        pltpu.SemaphoreType.DMA,
        pltpu.SemaphoreType.DMA,
    ],
)
def _onehot_sc(x_hbm, out_hbm, x_v0, x_v1, o_v0, o_v1,
               si0, si1, so0, so1):
    num_cores = 2
    b = lax.axis_index("s") * num_cores + lax.axis_index("c")
    x_bufs = (x_v0, x_v1)
    o_bufs = (o_v0, o_v1)
    in_sems = (si0, si1)
    out_sems = (so0, so1)

    def in_src(j):
        return x_hbm.at[b, pl.ds(j * C, C)]

    def out_dst(j, cls):
        return out_hbm.at[b, cls, pl.ds(j * C, C)]

    pltpu.async_copy(in_src(0), x_bufs[0], in_sems[0])
    pltpu.async_copy(in_src(1), x_bufs[1], in_sems[1])

    def pair_body(i, carry):
        for t in range(2):
            j = i * 2 + t
            x_v, o_v = x_bufs[t], o_bufs[t]
            pltpu.make_async_copy(in_src(j), x_v, in_sems[t]).wait()

            @pl.when(j >= 2)
            def _():
                for cls in range(NUM_CLASSES):
                    pltpu.make_async_copy(
                        o_v.at[cls], out_dst(j - 2, cls), out_sems[t]).wait()

            def vec_body(k, carry2):
                sl = pl.ds(k * LANES, LANES)
                v = x_v[sl]
                one = jnp.ones((LANES,), jnp.float32)
                zero = jnp.zeros((LANES,), jnp.float32)
                s0 = jnp.where(v < 0.1, one, zero)
                s1 = jnp.where(v < 1.0, one, zero)
                s2 = jnp.where(v < 2.5, one, zero)
                o_v[0, sl] = s0
                o_v[1, sl] = s1 - s0
                o_v[2, sl] = s2 - s1
                o_v[3, sl] = one - s2
                return carry2

            lax.fori_loop(0, C // LANES, vec_body, 0, unroll=4)

            for cls in range(NUM_CLASSES):
                pltpu.async_copy(o_v.at[cls], out_dst(j, cls), out_sems[t])

            @pl.when(j + 2 < NCHUNK)
            def _():
                pltpu.async_copy(in_src(j + 2), x_v, in_sems[t])
        return carry

    lax.fori_loop(0, NCHUNK // 2, pair_body, 0)

    for t in range(2):
        j = NCHUNK - 2 + t
        for cls in range(NUM_CLASSES):
            pltpu.make_async_copy(
                o_bufs[t].at[cls], out_dst(j, cls), out_sems[t]).wait()


def kernel(x):
    x2d = x.reshape(B, P)
    out = _onehot_sc(x2d)
    return out.reshape(B, 1, NUM_CLASSES, H, W)


# speedup vs baseline: 108.4267x; 1.1917x over previous
import functools

import jax
import jax.numpy as jnp
from jax import lax
from jax.experimental import pallas as pl
from jax.experimental.pallas import tpu as pltpu
from jax.experimental.pallas import tpu_sc as plsc

B, H, W = 32, 512, 512
P = H * W
NUM_CLASSES = 4
C = 8192
NCHUNK = P // C
LANES = 16

_mesh = plsc.VectorSubcoreMesh(core_axis_name="c", subcore_axis_name="s")


@functools.partial(
    pl.kernel,
    out_type=jax.ShapeDtypeStruct((B, NUM_CLASSES, P), jnp.float32),
    mesh=_mesh,
    scratch_types=[
        pltpu.VMEM((C,), jnp.float32),
        pltpu.VMEM((C,), jnp.float32),
        pltpu.VMEM((NUM_CLASSES, C), jnp.float32),
        pltpu.VMEM((NUM_CLASSES, C), jnp.float32),
        pltpu.SemaphoreType.DMA,
        pltpu.SemaphoreType.DMA,
        pltpu.SemaphoreType.DMA,
        pltpu.SemaphoreType.DMA,
    ],
)
def _onehot_sc(x_hbm, out_hbm, x_v0, x_v1, o_v0, o_v1,
               si0, si1, so0, so1):
    num_cores = 2
    b = lax.axis_index("s") * num_cores + lax.axis_index("c")
    x_bufs = (x_v0, x_v1)
    o_bufs = (o_v0, o_v1)
    in_sems = (si0, si1)
    out_sems = (so0, so1)

    def in_src(j):
        return x_hbm.at[b, pl.ds(j * C, C)]

    def out_dst(j, cls):
        return out_hbm.at[b, cls, pl.ds(j * C, C)]

    pltpu.async_copy(in_src(0), x_bufs[0], in_sems[0])
    pltpu.async_copy(in_src(1), x_bufs[1], in_sems[1])

    def pair_body(i, carry):
        for t in range(2):
            j = i * 2 + t
            x_v, o_v = x_bufs[t], o_bufs[t]
            pltpu.make_async_copy(in_src(j), x_v, in_sems[t]).wait()

            @pl.when(j >= 2)
            def _():
                for cls in range(NUM_CLASSES):
                    pltpu.make_async_copy(
                        o_v.at[cls], out_dst(j - 2, cls), out_sems[t]).wait()

            @plsc.parallel_loop(0, C, step=LANES, unroll=8)
            def _vec(k):
                sl = pl.ds(k, LANES)
                v = x_v[sl]
                one = jnp.ones((LANES,), jnp.float32)
                zero = jnp.zeros((LANES,), jnp.float32)
                s0 = jnp.where(v < 0.1, one, zero)
                s1 = jnp.where(v < 1.0, one, zero)
                s2 = jnp.where(v < 2.5, one, zero)
                o_v[0, sl] = s0
                o_v[1, sl] = s1 - s0
                o_v[2, sl] = s2 - s1
                o_v[3, sl] = one - s2

            for cls in range(NUM_CLASSES):
                pltpu.async_copy(o_v.at[cls], out_dst(j, cls), out_sems[t])

            @pl.when(j + 2 < NCHUNK)
            def _():
                pltpu.async_copy(in_src(j + 2), x_v, in_sems[t])
        return carry

    lax.fori_loop(0, NCHUNK // 2, pair_body, 0)

    for t in range(2):
        j = NCHUNK - 2 + t
        for cls in range(NUM_CLASSES):
            pltpu.make_async_copy(
                o_bufs[t].at[cls], out_dst(j, cls), out_sems[t]).wait()


def kernel(x):
    x2d = x.reshape(B, P)
    out = _onehot_sc(x2d)
    return out.reshape(B, 1, NUM_CLASSES, H, W)
